# baseline (speedup 1.0000x reference)
import functools

import jax
import jax.numpy as jnp
from jax import lax
from jax.experimental import pallas as pl
from jax.experimental.pallas import tpu as pltpu
from jax.experimental.pallas import tpu_sc as plsc

N_NODES = 10000
D = 128
NUM_GRAPHS = 64
NC = 2
NS = 16
NW = NC * NS
CH = 128

KD = 80
EPAD_D = NW * KD * CH

CP = 96
KP = 108
EPAD_P = NW * KP * CP

ZROWS = 10240
RPT = ZROWS // NS

_mesh = plsc.VectorSubcoreMesh(core_axis_name="c", subcore_axis_name="s")


@functools.partial(
    pl.kernel,
    out_type=jax.ShapeDtypeStruct((NC, ZROWS), jnp.float32),
    mesh=_mesh,
    scratch_types=[
        pltpu.VMEM((KD, CH), jnp.int32),
        pltpu.VMEM((KD, CH), jnp.float32),
        pltpu.VMEM((ZROWS,), jnp.float32),
        pltpu.VMEM_SHARED((ZROWS,), jnp.float32),
    ],
)
def _sc_deg(dst_hbm, w_hbm, deg_hbm, dst_v, w_v, zbuf, deg_sh):
    cid = lax.axis_index("c")
    sid = lax.axis_index("s")
    wid = cid * NS + sid

    @pl.when(sid == 0)
    def _():
        @pl.loop(0, ZROWS // 16)
        def _(i):
            zbuf[pl.ds(i * 16, 16)] = jnp.zeros((16,), jnp.float32)

        pltpu.sync_copy(zbuf, deg_sh)

    plsc.subcore_barrier()

    pltpu.sync_copy(dst_hbm.at[wid], dst_v)
    pltpu.sync_copy(w_hbm.at[wid], w_v)

    @pl.loop(0, KD)
    def _(j):
        pltpu.sync_copy(w_v.at[j], deg_sh.at[dst_v.at[j]], add=True)

    plsc.subcore_barrier()

    @pl.when(sid == 0)
    def _():
        pltpu.sync_copy(deg_sh, deg_hbm.at[cid])


def _tc_xw_body(x_ref, w_ref, xw_ref):
    xw_ref[...] = jnp.dot(x_ref[...], w_ref[...],
                          preferred_element_type=jnp.float32,
                          precision=lax.Precision.HIGHEST)


_tc_xw = pl.pallas_call(
    _tc_xw_body,
    out_shape=jax.ShapeDtypeStruct((N_NODES, D), jnp.float32),
)


@functools.partial(
    pl.kernel,
    out_type=jax.ShapeDtypeStruct((NC, ZROWS, D), jnp.float32),
    mesh=_mesh,
    compiler_params=pltpu.CompilerParams(use_tc_tiling_on_sc=False,
                                         needs_layout_passes=False),
    scratch_types=[
        pltpu.VMEM((3, CP), jnp.int32),
        pltpu.VMEM((3, 2, CP), jnp.int32),
        pltpu.VMEM((CP, D), jnp.float32),
        pltpu.VMEM((CP, D), jnp.float32),
        pltpu.VMEM((CP, D), jnp.float32),
        pltpu.VMEM((ZROWS,), jnp.float32),
        pltpu.VMEM_SHARED((ZROWS, D), jnp.float32),
        pltpu.SemaphoreType.DMA,
        pltpu.SemaphoreType.DMA,
        pltpu.SemaphoreType.DMA,
        pltpu.SemaphoreType.DMA,
        pltpu.SemaphoreType.DMA,
        pltpu.SemaphoreType.DMA,
        pltpu.SemaphoreType.DMA,
        pltpu.SemaphoreType.DMA,
        pltpu.SemaphoreType.DMA,
        pltpu.SemaphoreType.DMA,
        pltpu.SemaphoreType.DMA,
        pltpu.SemaphoreType.DMA,
    ],
)
def _sc_propagate(src_hbm, dw_hbm, degp_hbm, y_hbm, z_hbm,
                  src_r, dw_r, b0, b1, b2, dis_v, z_sh,
                  sg0, sg1, sg2, ss0, ss1, ss2,
                  sr0, sr1, sr2, sd0, sd1, sd2):
    cid = lax.axis_index("c")
    sid = lax.axis_index("s")
    wid = cid * NS + sid

    bufs = (b0, b1, b2)
    gsems = (sg0, sg1, sg2)
    ssems = (ss0, ss1, ss2)
    rsems = (sr0, sr1, sr2)
    dsems = (sd0, sd1, sd2)

    def issue_src(j, slot):
        pltpu.async_copy(src_hbm.at[wid, j], src_r.at[slot], rsems[slot])

    def wait_src(j, slot):
        pltpu.make_async_copy(
            src_hbm.at[wid, j], src_r.at[slot], rsems[slot]).wait()

    def issue_dw(j, slot):
        pltpu.async_copy(dw_hbm.at[wid, j], dw_r.at[slot], dsems[slot])

    def wait_dw(j, slot):
        pltpu.make_async_copy(
            dw_hbm.at[wid, j], dw_r.at[slot], dsems[slot]).wait()

    for s in range(3):
        issue_src(s, s)
    issue_dw(0, 0)
    issue_dw(1, 1)

    pltpu.sync_copy(degp_hbm.at[0], b0.at[pl.ds(0, ZROWS // D)])
    pltpu.sync_copy(degp_hbm.at[1], b1.at[pl.ds(0, ZROWS // D)])

    @pl.loop(0, ZROWS // D)
    def _(r):
        for c in range(D // 16):
            sl = pl.ds(c * 16, 16)
            dg = b0[r, sl] + b1[r, sl] + 1.0
            u = plsc.bitcast(dg, jnp.int32)
            g = plsc.bitcast(jnp.int32(0x5F3759DF) - (u >> 1), jnp.float32)
            for _it in range(3):
                g = g * (1.5 - 0.5 * dg * g * g)
            dis_v[pl.ds(r * D + c * 16, 16)] = g

    @pl.loop(0, CP)
    def _(r):
        for c in range(D // 16):
            b0[r, pl.ds(c * 16, 16)] = jnp.zeros((16,), jnp.float32)

    base = sid * RPT
    for t in range(RPT // CP):
        pltpu.sync_copy(b0, z_sh.at[pl.ds(base + t * CP, CP)])
    pltpu.sync_copy(b0.at[pl.ds(0, RPT - (RPT // CP) * CP)],
                    z_sh.at[pl.ds(base + (RPT // CP) * CP,
                                  RPT - (RPT // CP) * CP)])

    plsc.subcore_barrier()

    def scale(b, bv):
        @pl.loop(0, CP // 16)
        def _(g):
            w16 = plsc.bitcast(dw_r[b, 1, pl.ds(g * 16, 16)], jnp.float32)
            src16 = src_r[b, pl.ds(g * 16, 16)]
            m16 = w16 * plsc.load_gather(dis_v, [src16])
            for i in range(16):
                wr = m16[i]
                r = g * 16 + i
                for c in range(D // 16):
                    sl = pl.ds(c * 16, 16)
                    bv[r, sl] = bv[r, sl] * wr

    def bank(j, b, guarded, drain_prev):
        bv, gs = bufs[b], gsems[b]
        nb = (b + 2) % 3
        nbv = bufs[nb]
        pltpu.make_async_copy(y_hbm.at[src_r.at[b]], bv, gs).wait()

        wait_dw(j, b)
        scale(b, bv)

        def _src_next():
            issue_src(j + 3, b)
        (pl.when(j + 3 < KP)(_src_next) if guarded else _src_next())
        pltpu.async_copy(bv, z_sh.at[dw_r.at[b, 0]], ssems[b], add=True)

        def _next():
            if drain_prev:
                pltpu.make_async_copy(
                    nbv, z_sh.at[dw_r.at[b, 0]], ssems[nb]).wait()
            issue_dw(j + 2, nb)
            wait_src(j + 2, nb)
            pltpu.async_copy(y_hbm.at[src_r.at[nb]], nbv, gsems[nb])

        (pl.when(j + 2 < KP)(_next) if guarded else _next())

    wait_src(0, 0)
    pltpu.async_copy(y_hbm.at[src_r.at[0]], b0, sg0)
    wait_src(1, 1)
    pltpu.async_copy(y_hbm.at[src_r.at[1]], b1, sg1)
    bank(0, 0, False, False)
    bank(1, 1, False, True)
    bank(2, 2, False, True)

    @pl.loop(3, KP, step=3)
    def _(t):
        bank(t, 0, True, True)
        bank(t + 1, 1, True, True)
        bank(t + 2, 2, True, True)

    for b in range(3):
        pltpu.make_async_copy(bufs[b], z_sh.at[dw_r.at[b, 0]],
                              ssems[b]).wait()

    plsc.subcore_barrier()
    pltpu.sync_copy(z_sh.at[pl.ds(base, RPT)],
                    z_hbm.at[cid, pl.ds(base, RPT)])


def _tc_final_body(z_ref, xw_ref, degp_ref, b_ref, batch_ref, h_ref, g_ref):
    deg = degp_ref[:, 0:1] + degp_ref[:, 1:2] + 1.0
    dis = jnp.where(deg > 0, lax.rsqrt(deg), 0.0)
    z = z_ref[0, :N_NODES, :] + z_ref[1, :N_NODES, :]
    h = jnp.maximum(z * dis + xw_ref[...] * (dis * dis) + b_ref[...], 0.0)
    h_ref[...] = h
    iot = lax.broadcasted_iota(jnp.int32, (NUM_GRAPHS, N_NODES), 0)
    onehot = (batch_ref[...] == iot).astype(jnp.float32)
    counts = jnp.sum(onehot, axis=1, keepdims=True)
    sums = jnp.dot(onehot, h, preferred_element_type=jnp.float32,
                   precision=lax.Precision.HIGHEST)
    g_ref[...] = sums / jnp.maximum(counts, 1.0)


_tc_final = pl.pallas_call(
    _tc_final_body,
    out_shape=[
        jax.ShapeDtypeStruct((N_NODES, D), jnp.float32),
        jax.ShapeDtypeStruct((NUM_GRAPHS, D), jnp.float32),
    ],
)


def kernel(x, edge_index, edge_weight, batch, W, b):
    x = x.astype(jnp.float32)
    src = edge_index[0].astype(jnp.int32)
    dst = edge_index[1].astype(jnp.int32)
    w = edge_weight.astype(jnp.float32)
    e = src.shape[0]

    pad_d = (jnp.arange(EPAD_D - e, dtype=jnp.int32)
             % (ZROWS - N_NODES)) + N_NODES
    pad_p = (jnp.arange(EPAD_P - e, dtype=jnp.int32)
             % (ZROWS - N_NODES)) + N_NODES

    dst_d = jnp.concatenate([dst, pad_d]).reshape(NW, KD, CH)
    w_d = jnp.pad(w, (0, EPAD_D - e)).reshape(NW, KD, CH)

    pad_s = jnp.arange(EPAD_P - e, dtype=jnp.int32) % N_NODES
    src_p = jnp.concatenate([src, pad_s]).reshape(NW, KP, CP)
    dst_p = jnp.concatenate([dst, pad_p]).reshape(NW, KP, CP)
    w_p = jnp.pad(w, (0, EPAD_P - e)).reshape(NW, KP, CP)
    dw_p = jnp.stack(
        [dst_p, lax.bitcast_convert_type(w_p, jnp.int32)], axis=2)

    degp = _sc_deg(dst_d, w_d)
    degp_t = degp[:, :N_NODES].T
    degp_r = degp.reshape(NC, ZROWS // D, D)
    xw = _tc_xw(x, W.astype(jnp.float32))
    zp = _sc_propagate(src_p, dw_p, degp_r, xw)
    h, gemb = _tc_final(zp, xw, degp_t,
                        b.reshape(1, D).astype(jnp.float32),
                        batch.reshape(1, N_NODES).astype(jnp.int32))
    return (h, gemb)


# speedup vs baseline: 1.0235x; 1.0235x over previous
import functools

import jax
import jax.numpy as jnp
from jax import lax
from jax.experimental import pallas as pl
from jax.experimental.pallas import tpu as pltpu
from jax.experimental.pallas import tpu_sc as plsc

N_NODES = 10000
D = 128
NUM_GRAPHS = 64
NC = 2
NS = 16
NW = NC * NS
CH = 128

KD = 80
EPAD_D = NW * KD * CH

CP = 112
KP = 93
EPAD_P = NW * KP * CP

ZROWS = 10240
RPT = ZROWS // NS

_mesh = plsc.VectorSubcoreMesh(core_axis_name="c", subcore_axis_name="s")


@functools.partial(
    pl.kernel,
    out_type=jax.ShapeDtypeStruct((NC, ZROWS), jnp.float32),
    mesh=_mesh,
    scratch_types=[
        pltpu.VMEM((KD, CH), jnp.int32),
        pltpu.VMEM((KD, CH), jnp.float32),
        pltpu.VMEM((ZROWS,), jnp.float32),
        pltpu.VMEM_SHARED((ZROWS,), jnp.float32),
    ],
)
def _sc_deg(dst_hbm, w_hbm, deg_hbm, dst_v, w_v, zbuf, deg_sh):
    cid = lax.axis_index("c")
    sid = lax.axis_index("s")
    wid = cid * NS + sid

    @pl.when(sid == 0)
    def _():
        @pl.loop(0, ZROWS // 16)
        def _(i):
            zbuf[pl.ds(i * 16, 16)] = jnp.zeros((16,), jnp.float32)

        pltpu.sync_copy(zbuf, deg_sh)

    plsc.subcore_barrier()

    pltpu.sync_copy(dst_hbm.at[wid], dst_v)
    pltpu.sync_copy(w_hbm.at[wid], w_v)

    @pl.loop(0, KD)
    def _(j):
        pltpu.sync_copy(w_v.at[j], deg_sh.at[dst_v.at[j]], add=True)

    plsc.subcore_barrier()

    @pl.when(sid == 0)
    def _():
        pltpu.sync_copy(deg_sh, deg_hbm.at[cid])


def _tc_y_body(x_ref, w_ref, degp_ref, y_ref):
    deg = degp_ref[:, 0:1] + degp_ref[:, 1:2] + 1.0
    dis = jnp.where(deg > 0, lax.rsqrt(deg), 0.0)
    xw = jnp.dot(x_ref[...], w_ref[...],
                 preferred_element_type=jnp.float32,
                 precision=lax.Precision.HIGHEST)
    y_ref[...] = xw * dis


_tc_y = pl.pallas_call(
    _tc_y_body,
    out_shape=jax.ShapeDtypeStruct((N_NODES, D), jnp.float32),
)


@functools.partial(
    pl.kernel,
    out_type=jax.ShapeDtypeStruct((NC, ZROWS, D), jnp.float32),
    mesh=_mesh,
    compiler_params=pltpu.CompilerParams(use_tc_tiling_on_sc=False,
                                         needs_layout_passes=False),
    scratch_types=[
        pltpu.VMEM((3, CP), jnp.int32),
        pltpu.VMEM((3, 2, CP), jnp.int32),
        pltpu.VMEM((CP, D), jnp.float32),
        pltpu.VMEM((CP, D), jnp.float32),
        pltpu.VMEM((CP, D), jnp.float32),
        pltpu.VMEM_SHARED((ZROWS, D), jnp.float32),
        pltpu.SemaphoreType.DMA,
        pltpu.SemaphoreType.DMA,
        pltpu.SemaphoreType.DMA,
        pltpu.SemaphoreType.DMA,
        pltpu.SemaphoreType.DMA,
        pltpu.SemaphoreType.DMA,
        pltpu.SemaphoreType.DMA,
        pltpu.SemaphoreType.DMA,
        pltpu.SemaphoreType.DMA,
        pltpu.SemaphoreType.DMA,
        pltpu.SemaphoreType.DMA,
        pltpu.SemaphoreType.DMA,
    ],
)
def _sc_propagate(src_hbm, dw_hbm, y_hbm, z_hbm,
                  src_r, dw_r, b0, b1, b2, z_sh,
                  sg0, sg1, sg2, ss0, ss1, ss2,
                  sr0, sr1, sr2, sd0, sd1, sd2):
    cid = lax.axis_index("c")
    sid = lax.axis_index("s")
    wid = cid * NS + sid

    bufs = (b0, b1, b2)
    gsems = (sg0, sg1, sg2)
    ssems = (ss0, ss1, ss2)
    rsems = (sr0, sr1, sr2)
    dsems = (sd0, sd1, sd2)

    def issue_src(j, slot):
        pltpu.async_copy(src_hbm.at[wid, j], src_r.at[slot], rsems[slot])

    def wait_src(j, slot):
        pltpu.make_async_copy(
            src_hbm.at[wid, j], src_r.at[slot], rsems[slot]).wait()

    def issue_dw(j, slot):
        pltpu.async_copy(dw_hbm.at[wid, j], dw_r.at[slot], dsems[slot])

    def wait_dw(j, slot):
        pltpu.make_async_copy(
            dw_hbm.at[wid, j], dw_r.at[slot], dsems[slot]).wait()

    for s in range(3):
        issue_src(s, s)
    issue_dw(0, 0)
    issue_dw(1, 1)

    @pl.loop(0, CP)
    def _(r):
        for c in range(D // 16):
            b0[r, pl.ds(c * 16, 16)] = jnp.zeros((16,), jnp.float32)

    base = sid * RPT
    for t in range(RPT // CP):
        pltpu.sync_copy(b0, z_sh.at[pl.ds(base + t * CP, CP)])
    pltpu.sync_copy(b0.at[pl.ds(0, RPT - (RPT // CP) * CP)],
                    z_sh.at[pl.ds(base + (RPT // CP) * CP,
                                  RPT - (RPT // CP) * CP)])

    plsc.subcore_barrier()

    def scale(b, bv):
        @pl.loop(0, CP // 16)
        def _(g):
            w16 = plsc.bitcast(dw_r[b, 1, pl.ds(g * 16, 16)], jnp.float32)
            for i in range(16):
                wr = w16[i]
                r = g * 16 + i
                for c in range(D // 16):
                    sl = pl.ds(c * 16, 16)
                    bv[r, sl] = bv[r, sl] * wr

    def bank(j, b, guarded, drain_prev):
        bv, gs = bufs[b], gsems[b]
        nb = (b + 2) % 3
        nbv = bufs[nb]
        pltpu.make_async_copy(y_hbm.at[src_r.at[b]], bv, gs).wait()

        def _src_next():
            issue_src(j + 3, b)
        (pl.when(j + 3 < KP)(_src_next) if guarded else _src_next())

        wait_dw(j, b)
        scale(b, bv)
        pltpu.async_copy(bv, z_sh.at[dw_r.at[b, 0]], ssems[b], add=True)

        def _next():
            if drain_prev:
                pltpu.make_async_copy(
                    nbv, z_sh.at[dw_r.at[b, 0]], ssems[nb]).wait()
            issue_dw(j + 2, nb)
            wait_src(j + 2, nb)
            pltpu.async_copy(y_hbm.at[src_r.at[nb]], nbv, gsems[nb])

        (pl.when(j + 2 < KP)(_next) if guarded else _next())

    wait_src(0, 0)
    pltpu.async_copy(y_hbm.at[src_r.at[0]], b0, sg0)
    wait_src(1, 1)
    pltpu.async_copy(y_hbm.at[src_r.at[1]], b1, sg1)
    bank(0, 0, False, False)
    bank(1, 1, False, True)
    bank(2, 2, False, True)

    @pl.loop(3, KP, step=3)
    def _(t):
        bank(t, 0, True, True)
        bank(t + 1, 1, True, True)
        bank(t + 2, 2, True, True)

    for b in range(3):
        pltpu.make_async_copy(bufs[b], z_sh.at[dw_r.at[b, 0]],
                              ssems[b]).wait()

    plsc.subcore_barrier()
    pltpu.sync_copy(z_sh.at[pl.ds(base, RPT)],
                    z_hbm.at[cid, pl.ds(base, RPT)])


def _tc_final_body(z_ref, y_ref, degp_ref, b_ref, batch_ref, h_ref, g_ref):
    deg = degp_ref[:, 0:1] + degp_ref[:, 1:2] + 1.0
    dis = jnp.where(deg > 0, lax.rsqrt(deg), 0.0)
    z = z_ref[0, :N_NODES, :] + z_ref[1, :N_NODES, :]
    h = jnp.maximum((z + y_ref[...]) * dis + b_ref[...], 0.0)
    h_ref[...] = h
    iot = lax.broadcasted_iota(jnp.int32, (NUM_GRAPHS, N_NODES), 0)
    onehot = (batch_ref[...] == iot).astype(jnp.float32)
    counts = jnp.sum(onehot, axis=1, keepdims=True)
    sums = jnp.dot(onehot, h, preferred_element_type=jnp.float32,
                   precision=lax.Precision.HIGHEST)
    g_ref[...] = sums / jnp.maximum(counts, 1.0)


_tc_final = pl.pallas_call(
    _tc_final_body,
    out_shape=[
        jax.ShapeDtypeStruct((N_NODES, D), jnp.float32),
        jax.ShapeDtypeStruct((NUM_GRAPHS, D), jnp.float32),
    ],
)


def kernel(x, edge_index, edge_weight, batch, W, b):
    x = x.astype(jnp.float32)
    src = edge_index[0].astype(jnp.int32)
    dst = edge_index[1].astype(jnp.int32)
    w = edge_weight.astype(jnp.float32)
    e = src.shape[0]

    pad_d = (jnp.arange(EPAD_D - e, dtype=jnp.int32)
             % (ZROWS - N_NODES)) + N_NODES
    pad_p = (jnp.arange(EPAD_P - e, dtype=jnp.int32)
             % (ZROWS - N_NODES)) + N_NODES

    dst_d = jnp.concatenate([dst, pad_d]).reshape(NW, KD, CH)
    w_d = jnp.pad(w, (0, EPAD_D - e)).reshape(NW, KD, CH)

    pad_s = jnp.arange(EPAD_P - e, dtype=jnp.int32) % N_NODES
    src_p = jnp.concatenate([src, pad_s]).reshape(NW, KP, CP)
    dst_p = jnp.concatenate([dst, pad_p]).reshape(NW, KP, CP)
    w_p = jnp.pad(w, (0, EPAD_P - e)).reshape(NW, KP, CP)
    dw_p = jnp.stack(
        [dst_p, lax.bitcast_convert_type(w_p, jnp.int32)], axis=2)

    degp = _sc_deg(dst_d, w_d)
    degp_t = degp[:, :N_NODES].T
    y = _tc_y(x, W.astype(jnp.float32), degp_t)
    zp = _sc_propagate(src_p, dw_p, y)
    h, gemb = _tc_final(zp, y, degp_t,
                        b.reshape(1, D).astype(jnp.float32),
                        batch.reshape(1, N_NODES).astype(jnp.int32))
    return (h, gemb)
